# Initial kernel scaffold; baseline (speedup 1.0000x reference)
import jax
import jax.numpy as jnp
from jax.experimental import pallas as pl


def kernel(z, pos, batch, emb, Wf1, bf1, Wf2, Ws, Wm, Wout, shift):
    raise NotImplementedError("write your pallas kernel here")


# speedup vs baseline: 1.1976x; 1.1976x over previous
import functools

import jax
import jax.numpy as jnp
import numpy as np
from jax import lax
from jax.experimental import pallas as pl
from jax.experimental.pallas import tpu as pltpu
from jax.experimental.pallas import tpu_sc as plsc

_N = 10000
_K = 16
_NSP = 100
_NG = 64
_L = 32
_NRBF = 8
_RC = 6.0
_NB = 5
_E = _N * _K
_EP = 163840
_R = 1000
_GT = _N // _R
_ER = _R * _K


def _sc_gather(table, idx, d):
    B = idx.shape[0]
    nw = 32
    bpw = B // nw
    ch = 128
    nch = bpw // ch
    mesh = plsc.VectorSubcoreMesh(core_axis_name="c", subcore_axis_name="s")

    @functools.partial(
        pl.kernel,
        mesh=mesh,
        compiler_params=pltpu.CompilerParams(use_tc_tiling_on_sc=False),
        out_type=jax.ShapeDtypeStruct((B, d), jnp.float32),
        scratch_types=[
            pltpu.VMEM((ch,), jnp.int32),
            pltpu.VMEM((ch, d), jnp.float32),
            pltpu.SemaphoreType.DMA,
        ],
    )
    def k(table_hbm, idx_hbm, out_hbm, idx_v, rows_v, sem):
        wid = lax.axis_index("s") * 2 + lax.axis_index("c")

        def body(c, carry):
            base = wid * bpw + c * ch
            pltpu.sync_copy(idx_hbm.at[pl.ds(base, ch)], idx_v)
            pltpu.async_copy(table_hbm.at[idx_v], rows_v, sem).wait()
            pltpu.sync_copy(rows_v, out_hbm.at[pl.ds(base, ch)])
            return carry

        lax.fori_loop(0, nch, body, 0)

    return k(table, idx)


def _embed_rbf_body(z_ref, posw_ref, psg_ref, emb_ref, h_ref, rb_ref):
    z = z_ref[0, 0, :]
    oh = (z[:, None] == lax.broadcasted_iota(jnp.int32, (_R, _NSP), 1))
    h_ref[...] = jnp.dot(oh.astype(jnp.float32), emb_ref[...],
                         preferred_element_type=jnp.float32)
    ps = psg_ref[...].reshape(_R, _K, 16)
    rel = ps - posw_ref[...][:, None, :]
    d2 = jnp.sum(rel * rel, axis=-1) + 1e-12
    dist = jnp.sqrt(d2)
    dcl = jnp.maximum(dist, 1e-6)
    nvec = (lax.broadcasted_iota(jnp.int32, (_R, _K, _NRBF), 2) + 1
            ).astype(jnp.float32)
    basis = jnp.sin(nvec * (np.float32(np.pi) / _RC)
                    * dcl[:, :, None]) / dcl[:, :, None]
    env = 0.5 * (jnp.cos(np.float32(np.pi) * jnp.clip(dist, 0.0, _RC) / _RC)
                 + 1.0)
    rb_ref[...] = basis * env[:, :, None]


def _block_body(rb_ref, g_ref, h_ref, wf1_ref, bf1_ref, wf2_ref, ws_ref,
                wm_ref, ho_ref):
    rb = rb_ref[...].reshape(_ER, _NRBF)
    t = jnp.dot(rb, wf1_ref[...], preferred_element_type=jnp.float32)
    t = t + bf1_ref[...]
    t = t * jax.nn.sigmoid(t)
    w = jnp.dot(t, wf2_ref[...], preferred_element_type=jnp.float32)
    m = jnp.dot(g_ref[...], wm_ref[...],
                preferred_element_type=jnp.float32) * w
    agg = jnp.sum(m.reshape(_R, _K, _L), axis=1)
    s = jnp.dot(h_ref[...], ws_ref[...],
                preferred_element_type=jnp.float32) + agg
    ho_ref[...] = s * jax.nn.sigmoid(s)


def _out_body(h_ref, z_ref, b_ref, wout_ref, shift_ref, e_ref):
    i = pl.program_id(0)
    z = z_ref[0, 0, :]
    bb = b_ref[0, 0, :]
    ohz = (z[:, None] == lax.broadcasted_iota(jnp.int32, (_R, _NSP), 1))
    out = jnp.dot(h_ref[...], wout_ref[...],
                  preferred_element_type=jnp.float32)
    out = out + jnp.dot(ohz.astype(jnp.float32), shift_ref[...],
                        preferred_element_type=jnp.float32)
    ohbt = (lax.broadcasted_iota(jnp.int32, (_NG, _R), 0) == bb[None, :])
    part = jnp.dot(ohbt.astype(jnp.float32), out,
                   preferred_element_type=jnp.float32)

    @pl.when(i == 0)
    def _():
        e_ref[...] = jnp.zeros_like(e_ref)

    e_ref[...] += part


def _knn_src(pos, batch):
    sq = jnp.sum(pos * pos, axis=1)
    ch = 1000
    chunks = []
    for s in range(0, _N, ch):
        pc = pos[s:s + ch]
        d2 = (jnp.sum(pc * pc, axis=1)[:, None] + sq[None, :]
              - 2.0 * (pc @ pos.T))
        same = batch[s:s + ch, None] == batch[None, :]
        d2 = jnp.where(same, d2, 1e9)
        d2 = d2.at[jnp.arange(ch), jnp.arange(s, s + ch)].set(1e9)
        _, nbr = lax.top_k(-d2, _K)
        chunks.append(nbr)
    return jnp.concatenate(chunks, axis=0).reshape(-1)


def kernel(z, pos, batch, emb, Wf1, bf1, Wf2, Ws, Wm, Wout, shift):
    z = z.astype(jnp.int32)
    batch = batch.astype(jnp.int32)
    pos = pos.astype(jnp.float32)

    src = _knn_src(pos, batch).astype(jnp.int32)
    src_p = jnp.concatenate([src, jnp.zeros((_EP - _E,), jnp.int32)])

    posw = jnp.pad(pos, ((0, 0), (0, 13)))
    psg = _sc_gather(posw, src_p, 16)

    z3 = z.reshape(_GT, 1, _R)
    b3 = batch.reshape(_GT, 1, _R)

    h, rb = pl.pallas_call(
        _embed_rbf_body,
        grid=(_GT,),
        in_specs=[
            pl.BlockSpec((1, 1, _R), lambda i: (i, 0, 0)),
            pl.BlockSpec((_R, 16), lambda i: (i, 0)),
            pl.BlockSpec((_ER, 16), lambda i: (i, 0)),
            pl.BlockSpec((_NSP, _L), lambda i: (0, 0)),
        ],
        out_specs=[
            pl.BlockSpec((_R, _L), lambda i: (i, 0)),
            pl.BlockSpec((_R, _K, _NRBF), lambda i: (i, 0, 0)),
        ],
        out_shape=[
            jax.ShapeDtypeStruct((_N, _L), jnp.float32),
            jax.ShapeDtypeStruct((_N, _K, _NRBF), jnp.float32),
        ],
    )(z3, posw, psg, emb)

    for i in range(_NB):
        g = _sc_gather(h, src_p, _L)
        h = pl.pallas_call(
            _block_body,
            grid=(_GT,),
            in_specs=[
                pl.BlockSpec((_R, _K, _NRBF), lambda i: (i, 0, 0)),
                pl.BlockSpec((_ER, _L), lambda i: (i, 0)),
                pl.BlockSpec((_R, _L), lambda i: (i, 0)),
                pl.BlockSpec((_NRBF, _L), lambda i: (0, 0)),
                pl.BlockSpec((1, _L), lambda i: (0, 0)),
                pl.BlockSpec((_L, _L), lambda i: (0, 0)),
                pl.BlockSpec((_L, _L), lambda i: (0, 0)),
                pl.BlockSpec((_L, _L), lambda i: (0, 0)),
            ],
            out_specs=pl.BlockSpec((_R, _L), lambda i: (i, 0)),
            out_shape=jax.ShapeDtypeStruct((_N, _L), jnp.float32),
        )(rb, g, h, Wf1[i], bf1[i].reshape(1, _L), Wf2[i], Ws[i], Wm[i])

    energy = pl.pallas_call(
        _out_body,
        grid=(_GT,),
        in_specs=[
            pl.BlockSpec((_R, _L), lambda i: (i, 0)),
            pl.BlockSpec((1, 1, _R), lambda i: (i, 0, 0)),
            pl.BlockSpec((1, 1, _R), lambda i: (i, 0, 0)),
            pl.BlockSpec((_L, 1), lambda i: (0, 0)),
            pl.BlockSpec((_NSP, 1), lambda i: (0, 0)),
        ],
        out_specs=pl.BlockSpec((_NG, 1), lambda i: (0, 0)),
        out_shape=jax.ShapeDtypeStruct((_NG, 1), jnp.float32),
    )(h, z3, b3, Wout, shift)

    return energy


# speedup vs baseline: 1.6224x; 1.3547x over previous
import functools

import jax
import jax.numpy as jnp
import numpy as np
from jax import lax
from jax.experimental import pallas as pl
from jax.experimental.pallas import tpu as pltpu
from jax.experimental.pallas import tpu_sc as plsc

_N = 10000
_K = 16
_NSP = 100
_NG = 64
_L = 32
_NRBF = 8
_RC = 6.0
_NB = 5
_E = _N * _K
_EP = 163840
_R = 1000
_GT = _N // _R
_ER = _R * _K


def _sc_gather(table, idx, d):
    B = idx.shape[0]
    nw = 32
    bpw = B // nw
    ch = 128
    nch = bpw // ch
    mesh = plsc.VectorSubcoreMesh(core_axis_name="c", subcore_axis_name="s")

    @functools.partial(
        pl.kernel,
        mesh=mesh,
        compiler_params=pltpu.CompilerParams(use_tc_tiling_on_sc=False),
        out_type=jax.ShapeDtypeStruct((B, d), jnp.float32),
        scratch_types=[
            pltpu.VMEM((ch,), jnp.int32),
            pltpu.VMEM((ch, d), jnp.float32),
            pltpu.SemaphoreType.DMA,
        ],
    )
    def k(table_hbm, idx_hbm, out_hbm, idx_v, rows_v, sem):
        wid = lax.axis_index("s") * 2 + lax.axis_index("c")

        def body(c, carry):
            base = wid * bpw + c * ch
            pltpu.sync_copy(idx_hbm.at[pl.ds(base, ch)], idx_v)
            pltpu.async_copy(table_hbm.at[idx_v], rows_v, sem).wait()
            pltpu.sync_copy(rows_v, out_hbm.at[pl.ds(base, ch)])
            return carry

        lax.fori_loop(0, nch, body, 0)

    return k(table, idx)


def _embed_rbf_body(z_ref, posw_ref, psg_ref, emb_ref, h_ref, rb_ref):
    z = z_ref[0, 0, :]
    oh = (z[:, None] == lax.broadcasted_iota(jnp.int32, (_R, _NSP), 1))
    h_ref[...] = jnp.dot(oh.astype(jnp.float32), emb_ref[...],
                         preferred_element_type=jnp.float32)
    ps = psg_ref[...].reshape(_R, _K, 16)
    rel = ps - posw_ref[...][:, None, :]
    d2 = jnp.sum(rel * rel, axis=-1) + 1e-12
    dist = jnp.sqrt(d2)
    dcl = jnp.maximum(dist, 1e-6)
    nvec = (lax.broadcasted_iota(jnp.int32, (_R, _K, _NRBF), 2) + 1
            ).astype(jnp.float32)
    basis = jnp.sin(nvec * (np.float32(np.pi) / _RC)
                    * dcl[:, :, None]) / dcl[:, :, None]
    env = 0.5 * (jnp.cos(np.float32(np.pi) * jnp.clip(dist, 0.0, _RC) / _RC)
                 + 1.0)
    rb_ref[...] = basis * env[:, :, None]


def _block_body(rb_ref, g_ref, h_ref, wf1_ref, bf1_ref, wf2_ref, ws_ref,
                wm_ref, ho_ref):
    rb = rb_ref[...].reshape(_ER, _NRBF)
    t = jnp.dot(rb, wf1_ref[...], preferred_element_type=jnp.float32)
    t = t + bf1_ref[...]
    t = t * jax.nn.sigmoid(t)
    w = jnp.dot(t, wf2_ref[...], preferred_element_type=jnp.float32)
    m = jnp.dot(g_ref[...], wm_ref[...],
                preferred_element_type=jnp.float32) * w
    agg = jnp.sum(m.reshape(_R, _K, _L), axis=1)
    s = jnp.dot(h_ref[...], ws_ref[...],
                preferred_element_type=jnp.float32) + agg
    ho_ref[...] = s * jax.nn.sigmoid(s)


def _out_body(h_ref, z_ref, b_ref, wout_ref, shift_ref, e_ref):
    i = pl.program_id(0)
    z = z_ref[0, 0, :]
    bb = b_ref[0, 0, :]
    ohz = (z[:, None] == lax.broadcasted_iota(jnp.int32, (_R, _NSP), 1))
    out = jnp.dot(h_ref[...], wout_ref[...],
                  preferred_element_type=jnp.float32)
    out = out + jnp.dot(ohz.astype(jnp.float32), shift_ref[...],
                        preferred_element_type=jnp.float32)
    ohbt = (lax.broadcasted_iota(jnp.int32, (_NG, _R), 0) == bb[None, :])
    part = jnp.dot(ohbt.astype(jnp.float32), out,
                   preferred_element_type=jnp.float32)

    @pl.when(i == 0)
    def _():
        e_ref[...] = jnp.zeros_like(e_ref)

    e_ref[...] += part


def _knn_src(pos, batch):
    sq = jnp.sum(pos * pos, axis=1)
    ch = 1000
    w = 2048
    chunks = []
    for s in range(0, _N, ch):
        pc = pos[s:s + ch]
        pcsq = jnp.sum(pc * pc, axis=1)
        bc = batch[s:s + ch]
        rowidx = jnp.arange(s, s + ch)

        lo = jnp.searchsorted(batch, batch[s], side="left")
        hi = jnp.searchsorted(batch, batch[s + ch - 1], side="right")

        def fast(pc=pc, pcsq=pcsq, bc=bc, rowidx=rowidx, lo=lo, hi=hi):
            start = jnp.clip(lo, 0, _N - w)
            pw = lax.dynamic_slice(pos, (start, 0), (w, 3))
            bw = lax.dynamic_slice(batch, (start,), (w,))
            colidx = start + jnp.arange(w)
            d2 = pcsq[:, None] + jnp.sum(pw * pw, axis=1)[None, :] \
                - 2.0 * (pc @ pw.T)
            d2 = jnp.where(bc[:, None] == bw[None, :], d2, 1e9)
            d2 = jnp.where(rowidx[:, None] == colidx[None, :], 1e9, d2)
            _, nbr = lax.top_k(-d2, _K)
            return nbr + start

        def slow(pc=pc, pcsq=pcsq, bc=bc, rowidx=rowidx):
            d2 = pcsq[:, None] + sq[None, :] - 2.0 * (pc @ pos.T)
            d2 = jnp.where(bc[:, None] == batch[None, :], d2, 1e9)
            d2 = jnp.where(rowidx[:, None] == jnp.arange(_N)[None, :],
                           1e9, d2)
            _, nbr = lax.top_k(-d2, _K)
            return nbr

        chunks.append(lax.cond(hi - lo <= w, fast, slow))
    return jnp.concatenate(chunks, axis=0).reshape(-1)


def kernel(z, pos, batch, emb, Wf1, bf1, Wf2, Ws, Wm, Wout, shift):
    z = z.astype(jnp.int32)
    batch = batch.astype(jnp.int32)
    pos = pos.astype(jnp.float32)

    src = _knn_src(pos, batch).astype(jnp.int32)
    src_p = jnp.concatenate([src, jnp.zeros((_EP - _E,), jnp.int32)])

    posw = jnp.pad(pos, ((0, 0), (0, 13)))
    psg = _sc_gather(posw, src_p, 16)

    z3 = z.reshape(_GT, 1, _R)
    b3 = batch.reshape(_GT, 1, _R)

    h, rb = pl.pallas_call(
        _embed_rbf_body,
        grid=(_GT,),
        in_specs=[
            pl.BlockSpec((1, 1, _R), lambda i: (i, 0, 0)),
            pl.BlockSpec((_R, 16), lambda i: (i, 0)),
            pl.BlockSpec((_ER, 16), lambda i: (i, 0)),
            pl.BlockSpec((_NSP, _L), lambda i: (0, 0)),
        ],
        out_specs=[
            pl.BlockSpec((_R, _L), lambda i: (i, 0)),
            pl.BlockSpec((_R, _K, _NRBF), lambda i: (i, 0, 0)),
        ],
        out_shape=[
            jax.ShapeDtypeStruct((_N, _L), jnp.float32),
            jax.ShapeDtypeStruct((_N, _K, _NRBF), jnp.float32),
        ],
    )(z3, posw, psg, emb)

    for i in range(_NB):
        g = _sc_gather(h, src_p, _L)
        h = pl.pallas_call(
            _block_body,
            grid=(_GT,),
            in_specs=[
                pl.BlockSpec((_R, _K, _NRBF), lambda i: (i, 0, 0)),
                pl.BlockSpec((_ER, _L), lambda i: (i, 0)),
                pl.BlockSpec((_R, _L), lambda i: (i, 0)),
                pl.BlockSpec((_NRBF, _L), lambda i: (0, 0)),
                pl.BlockSpec((1, _L), lambda i: (0, 0)),
                pl.BlockSpec((_L, _L), lambda i: (0, 0)),
                pl.BlockSpec((_L, _L), lambda i: (0, 0)),
                pl.BlockSpec((_L, _L), lambda i: (0, 0)),
            ],
            out_specs=pl.BlockSpec((_R, _L), lambda i: (i, 0)),
            out_shape=jax.ShapeDtypeStruct((_N, _L), jnp.float32),
        )(rb, g, h, Wf1[i], bf1[i].reshape(1, _L), Wf2[i], Ws[i], Wm[i])

    energy = pl.pallas_call(
        _out_body,
        grid=(_GT,),
        in_specs=[
            pl.BlockSpec((_R, _L), lambda i: (i, 0)),
            pl.BlockSpec((1, 1, _R), lambda i: (i, 0, 0)),
            pl.BlockSpec((1, 1, _R), lambda i: (i, 0, 0)),
            pl.BlockSpec((_L, 1), lambda i: (0, 0)),
            pl.BlockSpec((_NSP, 1), lambda i: (0, 0)),
        ],
        out_specs=pl.BlockSpec((_NG, 1), lambda i: (0, 0)),
        out_shape=jax.ShapeDtypeStruct((_NG, 1), jnp.float32),
    )(h, z3, b3, Wout, shift)

    return energy


# speedup vs baseline: 2.4832x; 1.5305x over previous
import functools

import jax
import jax.numpy as jnp
import numpy as np
from jax import lax
from jax.experimental import pallas as pl
from jax.experimental.pallas import tpu as pltpu
from jax.experimental.pallas import tpu_sc as plsc

_N = 10000
_K = 16
_NSP = 100
_NG = 64
_L = 32
_NRBF = 8
_RC = 6.0
_NB = 5
_E = _N * _K
_EP = 163840
_R = 1000
_GT = _N // _R
_ER = _R * _K


def _sc_gather(table, idx, d):
    B = idx.shape[0]
    nw = 32
    bpw = B // nw
    ch = 128
    nch = bpw // ch
    mesh = plsc.VectorSubcoreMesh(core_axis_name="c", subcore_axis_name="s")

    @functools.partial(
        pl.kernel,
        mesh=mesh,
        compiler_params=pltpu.CompilerParams(use_tc_tiling_on_sc=False),
        out_type=jax.ShapeDtypeStruct((B, d), jnp.float32),
        scratch_types=[
            pltpu.VMEM((ch,), jnp.int32),
            pltpu.VMEM((ch, d), jnp.float32),
            pltpu.SemaphoreType.DMA,
        ],
    )
    def k(table_hbm, idx_hbm, out_hbm, idx_v, rows_v, sem):
        wid = lax.axis_index("s") * 2 + lax.axis_index("c")

        def body(c, carry):
            base = wid * bpw + c * ch
            pltpu.sync_copy(idx_hbm.at[pl.ds(base, ch)], idx_v)
            pltpu.async_copy(table_hbm.at[idx_v], rows_v, sem).wait()
            pltpu.sync_copy(rows_v, out_hbm.at[pl.ds(base, ch)])
            return carry

        lax.fori_loop(0, nch, body, 0)

    return k(table, idx)


def _embed_rbf_body(z_ref, posw_ref, psg_ref, emb_ref, h_ref, rb_ref):
    z = z_ref[0, 0, :]
    oh = (z[:, None] == lax.broadcasted_iota(jnp.int32, (_R, _NSP), 1))
    h_ref[...] = jnp.dot(oh.astype(jnp.float32), emb_ref[...],
                         preferred_element_type=jnp.float32)
    ps = psg_ref[...].reshape(_R, _K, 16)
    rel = ps - posw_ref[...][:, None, :]
    d2 = jnp.sum(rel * rel, axis=-1) + 1e-12
    dist = jnp.sqrt(d2)
    dcl = jnp.maximum(dist, 1e-6)
    nvec = (lax.broadcasted_iota(jnp.int32, (_R, _K, _NRBF), 2) + 1
            ).astype(jnp.float32)
    basis = jnp.sin(nvec * (np.float32(np.pi) / _RC)
                    * dcl[:, :, None]) / dcl[:, :, None]
    env = 0.5 * (jnp.cos(np.float32(np.pi) * jnp.clip(dist, 0.0, _RC) / _RC)
                 + 1.0)
    rb_ref[...] = basis * env[:, :, None]


def _block_body(rb_ref, g_ref, h_ref, wf1_ref, bf1_ref, wf2_ref, ws_ref,
                wm_ref, ho_ref):
    rb = rb_ref[...].reshape(_ER, _NRBF)
    t = jnp.dot(rb, wf1_ref[...], preferred_element_type=jnp.float32)
    t = t + bf1_ref[...]
    t = t * jax.nn.sigmoid(t)
    w = jnp.dot(t, wf2_ref[...], preferred_element_type=jnp.float32)
    m = jnp.dot(g_ref[...], wm_ref[...],
                preferred_element_type=jnp.float32) * w
    agg = jnp.sum(m.reshape(_R, _K, _L), axis=1)
    s = jnp.dot(h_ref[...], ws_ref[...],
                preferred_element_type=jnp.float32) + agg
    ho_ref[...] = s * jax.nn.sigmoid(s)


def _out_body(h_ref, z_ref, b_ref, wout_ref, shift_ref, e_ref):
    i = pl.program_id(0)
    z = z_ref[0, 0, :]
    bb = b_ref[0, 0, :]
    ohz = (z[:, None] == lax.broadcasted_iota(jnp.int32, (_R, _NSP), 1))
    out = jnp.dot(h_ref[...], wout_ref[...],
                  preferred_element_type=jnp.float32)
    out = out + jnp.dot(ohz.astype(jnp.float32), shift_ref[...],
                        preferred_element_type=jnp.float32)
    ohbt = (lax.broadcasted_iota(jnp.int32, (_NG, _R), 0) == bb[None, :])
    part = jnp.dot(ohbt.astype(jnp.float32), out,
                   preferred_element_type=jnp.float32)

    @pl.when(i == 0)
    def _():
        e_ref[...] = jnp.zeros_like(e_ref)

    e_ref[...] += part


def _knn_src(pos, batch):
    sq = jnp.sum(pos * pos, axis=1)
    ch = 500
    w = 1024
    chunks = []
    for s in range(0, _N, ch):
        pc = pos[s:s + ch]
        pcsq = jnp.sum(pc * pc, axis=1)
        bc = batch[s:s + ch]
        rowidx = jnp.arange(s, s + ch)

        lo = jnp.searchsorted(batch, batch[s], side="left")
        hi = jnp.searchsorted(batch, batch[s + ch - 1], side="right")

        def fast(pc=pc, pcsq=pcsq, bc=bc, rowidx=rowidx, lo=lo, hi=hi):
            start = jnp.clip(lo, 0, _N - w)
            pw = lax.dynamic_slice(pos, (start, 0), (w, 3))
            bw = lax.dynamic_slice(batch, (start,), (w,))
            colidx = start + jnp.arange(w)
            d2 = pcsq[:, None] + jnp.sum(pw * pw, axis=1)[None, :] \
                - 2.0 * (pc @ pw.T)
            d2 = jnp.where(bc[:, None] == bw[None, :], d2, 1e9)
            d2 = jnp.where(rowidx[:, None] == colidx[None, :], 1e9, d2)
            _, nbr = lax.top_k(-d2, _K)
            return nbr + start

        def slow(pc=pc, pcsq=pcsq, bc=bc, rowidx=rowidx):
            d2 = pcsq[:, None] + sq[None, :] - 2.0 * (pc @ pos.T)
            d2 = jnp.where(bc[:, None] == batch[None, :], d2, 1e9)
            d2 = jnp.where(rowidx[:, None] == jnp.arange(_N)[None, :],
                           1e9, d2)
            _, nbr = lax.top_k(-d2, _K)
            return nbr

        chunks.append(lax.cond(hi - lo <= w, fast, slow))
    return jnp.concatenate(chunks, axis=0).reshape(-1)


def kernel(z, pos, batch, emb, Wf1, bf1, Wf2, Ws, Wm, Wout, shift):
    z = z.astype(jnp.int32)
    batch = batch.astype(jnp.int32)
    pos = pos.astype(jnp.float32)

    src = _knn_src(pos, batch).astype(jnp.int32)
    src_p = jnp.concatenate([src, jnp.zeros((_EP - _E,), jnp.int32)])

    posw = jnp.pad(pos, ((0, 0), (0, 13)))
    psg = _sc_gather(posw, src_p, 16)

    z3 = z.reshape(_GT, 1, _R)
    b3 = batch.reshape(_GT, 1, _R)

    h, rb = pl.pallas_call(
        _embed_rbf_body,
        grid=(_GT,),
        in_specs=[
            pl.BlockSpec((1, 1, _R), lambda i: (i, 0, 0)),
            pl.BlockSpec((_R, 16), lambda i: (i, 0)),
            pl.BlockSpec((_ER, 16), lambda i: (i, 0)),
            pl.BlockSpec((_NSP, _L), lambda i: (0, 0)),
        ],
        out_specs=[
            pl.BlockSpec((_R, _L), lambda i: (i, 0)),
            pl.BlockSpec((_R, _K, _NRBF), lambda i: (i, 0, 0)),
        ],
        out_shape=[
            jax.ShapeDtypeStruct((_N, _L), jnp.float32),
            jax.ShapeDtypeStruct((_N, _K, _NRBF), jnp.float32),
        ],
    )(z3, posw, psg, emb)

    for i in range(_NB):
        g = _sc_gather(h, src_p, _L)
        h = pl.pallas_call(
            _block_body,
            grid=(_GT,),
            in_specs=[
                pl.BlockSpec((_R, _K, _NRBF), lambda i: (i, 0, 0)),
                pl.BlockSpec((_ER, _L), lambda i: (i, 0)),
                pl.BlockSpec((_R, _L), lambda i: (i, 0)),
                pl.BlockSpec((_NRBF, _L), lambda i: (0, 0)),
                pl.BlockSpec((1, _L), lambda i: (0, 0)),
                pl.BlockSpec((_L, _L), lambda i: (0, 0)),
                pl.BlockSpec((_L, _L), lambda i: (0, 0)),
                pl.BlockSpec((_L, _L), lambda i: (0, 0)),
            ],
            out_specs=pl.BlockSpec((_R, _L), lambda i: (i, 0)),
            out_shape=jax.ShapeDtypeStruct((_N, _L), jnp.float32),
        )(rb, g, h, Wf1[i], bf1[i].reshape(1, _L), Wf2[i], Ws[i], Wm[i])

    energy = pl.pallas_call(
        _out_body,
        grid=(_GT,),
        in_specs=[
            pl.BlockSpec((_R, _L), lambda i: (i, 0)),
            pl.BlockSpec((1, 1, _R), lambda i: (i, 0, 0)),
            pl.BlockSpec((1, 1, _R), lambda i: (i, 0, 0)),
            pl.BlockSpec((_L, 1), lambda i: (0, 0)),
            pl.BlockSpec((_NSP, 1), lambda i: (0, 0)),
        ],
        out_specs=pl.BlockSpec((_NG, 1), lambda i: (0, 0)),
        out_shape=jax.ShapeDtypeStruct((_NG, 1), jnp.float32),
    )(h, z3, b3, Wout, shift)

    return energy


# speedup vs baseline: 2.5404x; 1.0231x over previous
import functools

import jax
import jax.numpy as jnp
import numpy as np
from jax import lax
from jax.experimental import pallas as pl
from jax.experimental.pallas import tpu as pltpu
from jax.experimental.pallas import tpu_sc as plsc

_N = 10000
_K = 16
_NSP = 100
_NG = 64
_L = 32
_NRBF = 8
_RC = 6.0
_NB = 5
_E = _N * _K
_EP = 163840
_R = 1000
_GT = _N // _R
_ER = _R * _K


def _sc_gather(table, idx, d):
    B = idx.shape[0]
    nw = 32
    bpw = B // nw
    ch = 512
    nch = bpw // ch
    mesh = plsc.VectorSubcoreMesh(core_axis_name="c", subcore_axis_name="s")

    @functools.partial(
        pl.kernel,
        mesh=mesh,
        compiler_params=pltpu.CompilerParams(use_tc_tiling_on_sc=False),
        out_type=jax.ShapeDtypeStruct((B, d), jnp.float32),
        scratch_types=[
            pltpu.VMEM((ch,), jnp.int32),
            pltpu.VMEM((ch, d), jnp.float32),
            pltpu.SemaphoreType.DMA,
        ],
    )
    def k(table_hbm, idx_hbm, out_hbm, idx_v, rows_v, sem):
        wid = lax.axis_index("s") * 2 + lax.axis_index("c")

        def body(c, carry):
            base = wid * bpw + c * ch
            pltpu.sync_copy(idx_hbm.at[pl.ds(base, ch)], idx_v)
            pltpu.async_copy(table_hbm.at[idx_v], rows_v, sem).wait()
            pltpu.sync_copy(rows_v, out_hbm.at[pl.ds(base, ch)])
            return carry

        lax.fori_loop(0, nch, body, 0)

    return k(table, idx)


def _embed_rbf_body(z_ref, posw_ref, psg_ref, emb_ref, h_ref, rb_ref):
    z = z_ref[0, 0, :]
    oh = (z[:, None] == lax.broadcasted_iota(jnp.int32, (_R, _NSP), 1))
    h_ref[...] = jnp.dot(oh.astype(jnp.float32), emb_ref[...],
                         preferred_element_type=jnp.float32)
    ps = psg_ref[...].reshape(_R, _K, 16)
    rel = ps - posw_ref[...][:, None, :]
    d2 = jnp.sum(rel * rel, axis=-1) + 1e-12
    dist = jnp.sqrt(d2)
    dcl = jnp.maximum(dist, 1e-6)
    nvec = (lax.broadcasted_iota(jnp.int32, (_R, _K, _NRBF), 2) + 1
            ).astype(jnp.float32)
    basis = jnp.sin(nvec * (np.float32(np.pi) / _RC)
                    * dcl[:, :, None]) / dcl[:, :, None]
    env = 0.5 * (jnp.cos(np.float32(np.pi) * jnp.clip(dist, 0.0, _RC) / _RC)
                 + 1.0)
    rb_ref[...] = basis * env[:, :, None]


def _block_body(rb_ref, g_ref, h_ref, wf1_ref, bf1_ref, wf2_ref, ws_ref,
                wm_ref, ho_ref):
    rb = rb_ref[...].reshape(_ER, _NRBF)
    t = jnp.dot(rb, wf1_ref[...], preferred_element_type=jnp.float32)
    t = t + bf1_ref[...]
    t = t * jax.nn.sigmoid(t)
    w = jnp.dot(t, wf2_ref[...], preferred_element_type=jnp.float32)
    m = jnp.dot(g_ref[...], wm_ref[...],
                preferred_element_type=jnp.float32) * w
    agg = jnp.sum(m.reshape(_R, _K, _L), axis=1)
    s = jnp.dot(h_ref[...], ws_ref[...],
                preferred_element_type=jnp.float32) + agg
    ho_ref[...] = s * jax.nn.sigmoid(s)


def _out_body(h_ref, z_ref, b_ref, wout_ref, shift_ref, e_ref):
    i = pl.program_id(0)
    z = z_ref[0, 0, :]
    bb = b_ref[0, 0, :]
    ohz = (z[:, None] == lax.broadcasted_iota(jnp.int32, (_R, _NSP), 1))
    out = jnp.dot(h_ref[...], wout_ref[...],
                  preferred_element_type=jnp.float32)
    out = out + jnp.dot(ohz.astype(jnp.float32), shift_ref[...],
                        preferred_element_type=jnp.float32)
    ohbt = (lax.broadcasted_iota(jnp.int32, (_NG, _R), 0) == bb[None, :])
    part = jnp.dot(ohbt.astype(jnp.float32), out,
                   preferred_element_type=jnp.float32)

    @pl.when(i == 0)
    def _():
        e_ref[...] = jnp.zeros_like(e_ref)

    e_ref[...] += part


def _knn_src(pos, batch):
    sq = jnp.sum(pos * pos, axis=1)
    ch = 500
    w = 1024
    chunks = []
    for s in range(0, _N, ch):
        pc = pos[s:s + ch]
        pcsq = jnp.sum(pc * pc, axis=1)
        bc = batch[s:s + ch]
        rowidx = jnp.arange(s, s + ch)

        lo = jnp.searchsorted(batch, batch[s], side="left")
        hi = jnp.searchsorted(batch, batch[s + ch - 1], side="right")

        def fast(pc=pc, pcsq=pcsq, bc=bc, rowidx=rowidx, lo=lo, hi=hi):
            start = jnp.clip(lo, 0, _N - w)
            pw = lax.dynamic_slice(pos, (start, 0), (w, 3))
            bw = lax.dynamic_slice(batch, (start,), (w,))
            colidx = start + jnp.arange(w)
            d2 = pcsq[:, None] + jnp.sum(pw * pw, axis=1)[None, :] \
                - 2.0 * (pc @ pw.T)
            d2 = jnp.where(bc[:, None] == bw[None, :], d2, 1e9)
            d2 = jnp.where(rowidx[:, None] == colidx[None, :], 1e9, d2)
            _, nbr = lax.top_k(-d2, _K)
            return nbr + start

        def slow(pc=pc, pcsq=pcsq, bc=bc, rowidx=rowidx):
            d2 = pcsq[:, None] + sq[None, :] - 2.0 * (pc @ pos.T)
            d2 = jnp.where(bc[:, None] == batch[None, :], d2, 1e9)
            d2 = jnp.where(rowidx[:, None] == jnp.arange(_N)[None, :],
                           1e9, d2)
            _, nbr = lax.top_k(-d2, _K)
            return nbr

        chunks.append(lax.cond(hi - lo <= w, fast, slow))
    return jnp.concatenate(chunks, axis=0).reshape(-1)


def kernel(z, pos, batch, emb, Wf1, bf1, Wf2, Ws, Wm, Wout, shift):
    z = z.astype(jnp.int32)
    batch = batch.astype(jnp.int32)
    pos = pos.astype(jnp.float32)

    src = _knn_src(pos, batch).astype(jnp.int32)
    src_p = jnp.concatenate([src, jnp.zeros((_EP - _E,), jnp.int32)])

    posw = jnp.pad(pos, ((0, 0), (0, 13)))
    psg = _sc_gather(posw, src_p, 16)

    z3 = z.reshape(_GT, 1, _R)
    b3 = batch.reshape(_GT, 1, _R)

    h, rb = pl.pallas_call(
        _embed_rbf_body,
        grid=(_GT,),
        in_specs=[
            pl.BlockSpec((1, 1, _R), lambda i: (i, 0, 0)),
            pl.BlockSpec((_R, 16), lambda i: (i, 0)),
            pl.BlockSpec((_ER, 16), lambda i: (i, 0)),
            pl.BlockSpec((_NSP, _L), lambda i: (0, 0)),
        ],
        out_specs=[
            pl.BlockSpec((_R, _L), lambda i: (i, 0)),
            pl.BlockSpec((_R, _K, _NRBF), lambda i: (i, 0, 0)),
        ],
        out_shape=[
            jax.ShapeDtypeStruct((_N, _L), jnp.float32),
            jax.ShapeDtypeStruct((_N, _K, _NRBF), jnp.float32),
        ],
    )(z3, posw, psg, emb)

    for i in range(_NB):
        g = _sc_gather(h, src_p, _L)
        h = pl.pallas_call(
            _block_body,
            grid=(_GT,),
            in_specs=[
                pl.BlockSpec((_R, _K, _NRBF), lambda i: (i, 0, 0)),
                pl.BlockSpec((_ER, _L), lambda i: (i, 0)),
                pl.BlockSpec((_R, _L), lambda i: (i, 0)),
                pl.BlockSpec((_NRBF, _L), lambda i: (0, 0)),
                pl.BlockSpec((1, _L), lambda i: (0, 0)),
                pl.BlockSpec((_L, _L), lambda i: (0, 0)),
                pl.BlockSpec((_L, _L), lambda i: (0, 0)),
                pl.BlockSpec((_L, _L), lambda i: (0, 0)),
            ],
            out_specs=pl.BlockSpec((_R, _L), lambda i: (i, 0)),
            out_shape=jax.ShapeDtypeStruct((_N, _L), jnp.float32),
        )(rb, g, h, Wf1[i], bf1[i].reshape(1, _L), Wf2[i], Ws[i], Wm[i])

    energy = pl.pallas_call(
        _out_body,
        grid=(_GT,),
        in_specs=[
            pl.BlockSpec((_R, _L), lambda i: (i, 0)),
            pl.BlockSpec((1, 1, _R), lambda i: (i, 0, 0)),
            pl.BlockSpec((1, 1, _R), lambda i: (i, 0, 0)),
            pl.BlockSpec((_L, 1), lambda i: (0, 0)),
            pl.BlockSpec((_NSP, 1), lambda i: (0, 0)),
        ],
        out_specs=pl.BlockSpec((_NG, 1), lambda i: (0, 0)),
        out_shape=jax.ShapeDtypeStruct((_NG, 1), jnp.float32),
    )(h, z3, b3, Wout, shift)

    return energy
